# baseline (speedup 1.0000x reference)
import functools

import jax
import jax.numpy as jnp
from jax import lax
from jax.experimental import pallas as pl
from jax.experimental.pallas import tpu as pltpu
from jax.experimental.pallas import tpu_sc as plsc

BEAM = 4
VOCAB = 100000
NKV = 16
NEG_BIG = -1e30
BIGI = jnp.int32(2 ** 30)
NSLOTS = 5

L = 16
LPAD = 102400
LCHUNK = LPAD // 16
LVREGS = LCHUNK // L

RP_W = 100096
RP_FLAT = BEAM * RP_W
RP_CHUNK = RP_FLAT // 32
RP_VREGS = RP_CHUNK // L

_LN2 = 0.6931471805599453


def _sc_body(logits_hbm, pen_hbm, rp_hbm,
             out_i_hbm, out_v_hbm, rp_out_hbm,
             lbuf, rbuf, pen_v, row_v, irow_v, m2, i2,
             stage_v, stage_i):
    cid = lax.axis_index("c")
    sid = lax.axis_index("s")
    lane = lax.broadcasted_iota(jnp.int32, (L,), 0)

    pltpu.sync_copy(pen_hbm, pen_v)

    base_l = sid * LCHUNK
    pltpu.sync_copy(logits_hbm.at[pl.ds(base_l, LCHUNK)], lbuf)

    def pass1(j, c):
        bv, bi = c
        v = lbuf[pl.ds(j * L, L)]
        gi = base_l + j * L + lane
        upd = v > bv
        return jnp.where(upd, v, bv), jnp.where(upd, gi, bi)

    bv, bi = lax.fori_loop(
        0, LVREGS, pass1,
        (jnp.full((L,), NEG_BIG, jnp.float32), jnp.zeros((L,), jnp.int32)))
    m1 = jnp.max(bv)
    i1 = jnp.min(jnp.where(bv == m1, bi, BIGI))
    m_w = m1

    def pass_se(j, s):
        v = lbuf[pl.ds(j * L, L)]
        return s + jnp.exp(v - m_w)

    s_vec = lax.fori_loop(0, LVREGS, pass_se, jnp.zeros((L,), jnp.float32))
    s_w = jnp.sum(s_vec)

    found_v = [m1]
    found_i = [i1]
    for _ in range(1, BEAM):
        prev = list(found_i)

        def passt(j, c, prev=prev):
            bv2, bi2 = c
            v = lbuf[pl.ds(j * L, L)]
            gi = base_l + j * L + lane
            excl = gi == prev[0]
            for p in prev[1:]:
                excl = excl | (gi == p)
            v = jnp.where(excl, NEG_BIG, v)
            upd = v > bv2
            return jnp.where(upd, v, bv2), jnp.where(upd, gi, bi2)

        bv2, bi2 = lax.fori_loop(
            0, LVREGS, passt,
            (jnp.full((L,), NEG_BIG, jnp.float32), jnp.zeros((L,), jnp.int32)))
        mt = jnp.max(bv2)
        it = jnp.min(jnp.where(bv2 == mt, bi2, BIGI))
        found_v.append(mt)
        found_i.append(it)

    sv = jnp.zeros((L,), jnp.float32)
    si = jnp.zeros((L,), jnp.int32)
    for t in range(BEAM):
        sv = jnp.where(lane == t, found_v[t], sv)
        si = jnp.where(lane == t, found_i[t], si)
    sv = jnp.where(lane == BEAM, m_w, sv)
    sv = jnp.where(lane == BEAM + 1, s_w, sv)
    row_v[...] = sv
    irow_v[...] = si
    pltpu.sync_copy(row_v, stage_v.at[pl.ds(sid * L, L)])
    pltpu.sync_copy(irow_v, stage_i.at[pl.ds(sid * L, L)])
    plsc.subcore_barrier()

    @pl.when(sid == 0)
    def _merge():
        pltpu.sync_copy(stage_v, m2)
        pltpu.sync_copy(stage_i, i2)

        def gmax(r, acc):
            rowr = m2[pl.ds(r * L, L)]
            m_r = jnp.max(jnp.where(lane == BEAM, rowr, NEG_BIG))
            return jnp.maximum(acc, m_r)

        gm = lax.fori_loop(0, 16, gmax, jnp.float32(NEG_BIG))

        def gsum(r, acc):
            rowr = m2[pl.ds(r * L, L)]
            m_r = jnp.max(jnp.where(lane == BEAM, rowr, NEG_BIG))
            s_r = jnp.max(jnp.where(lane == BEAM + 1, rowr, NEG_BIG))
            e = jnp.exp((m_r - gm) + jnp.zeros((L,), jnp.float32))
            return acc + s_r * jnp.max(e)

        gs = lax.fori_loop(0, 16, gsum, jnp.float32(0.0))

        g_v = []
        g_i = []
        for _ in range(BEAM):
            prev = list(g_i)

            def passg(r, c, prev=prev):
                bv3, bi3 = c
                vrow = m2[pl.ds(r * L, L)]
                irow = i2[pl.ds(r * L, L)]
                cand = jnp.where(lane < BEAM, vrow, NEG_BIG)
                for p in prev:
                    cand = jnp.where(irow == p, NEG_BIG, cand)
                upd = cand > bv3
                return jnp.where(upd, cand, bv3), jnp.where(upd, irow, bi3)

            bv3, bi3 = lax.fori_loop(
                0, 16, passg,
                (jnp.full((L,), NEG_BIG, jnp.float32),
                 jnp.zeros((L,), jnp.int32)))
            mt = jnp.max(bv3)
            it = jnp.min(jnp.where(bv3 == mt, bi3, BIGI))
            g_v.append(mt)
            g_i.append(it)

        s_splat = gs + jnp.zeros((L,), jnp.float32)
        bits = lax.bitcast_convert_type(s_splat, jnp.int32)
        e0 = ((bits >> 23) & 255) - 127
        y = e0.astype(jnp.float32) * jnp.float32(_LN2)
        for _ in range(4):
            y = y - 1.0 + s_splat * jnp.exp(-y)
        lse = gm + jnp.max(y)

        ov = jnp.zeros((L,), jnp.float32)
        oi = jnp.zeros((L,), jnp.int32)
        for t in range(BEAM):
            ov = jnp.where(lane == t, g_v[t] - lse, ov)
            oi = jnp.where(lane == t, g_i[t], oi)
        row_v[...] = ov
        irow_v[...] = oi
        pltpu.sync_copy(irow_v, stage_i.at[pl.ds(0, L)])

        @pl.when(cid == 0)
        def _write_small():
            pltpu.sync_copy(row_v, out_v_hbm)
            pltpu.sync_copy(irow_v, out_i_hbm)

    plsc.subcore_barrier()

    pltpu.sync_copy(stage_i.at[pl.ds(0, L)], irow_v)
    irow = irow_v[...]
    j0 = jnp.max(jnp.where(lane == 0, irow, -1))
    j1 = jnp.max(jnp.where(lane == 1, irow, -1))
    j2 = jnp.max(jnp.where(lane == 2, irow, -1))
    j3 = jnp.max(jnp.where(lane == 3, irow, -1))
    pen = pen_v[...]

    chunk_id = sid * 2 + cid
    base_r = chunk_id * RP_CHUNK
    pltpu.sync_copy(rp_hbm.at[pl.ds(base_r, RP_CHUNK)], rbuf)

    def rp_pass(j, carry):
        v = rbuf[pl.ds(j * L, L)]
        g = base_r + j * L + lane
        colx = lax.rem(g, RP_W)
        hit = (colx == j0) | (colx == j1) | (colx == j2) | (colx == j3)
        rbuf[pl.ds(j * L, L)] = jnp.where(hit, v * pen, v)
        return carry

    lax.fori_loop(0, RP_VREGS, rp_pass, 0)
    pltpu.sync_copy(rbuf, rp_out_hbm.at[pl.ds(base_r, RP_CHUNK)])


def _sc_topk(logits, repeat_penality, penality_value):
    lflat = jnp.concatenate(
        [logits.reshape(VOCAB),
         jnp.full((LPAD - VOCAB,), NEG_BIG, jnp.float32)])
    pen16 = jnp.tile(penality_value, L)
    rp_pad = jnp.pad(repeat_penality, ((0, 0), (0, RP_W - VOCAB)),
                     constant_values=1.0).reshape(RP_FLAT)

    mesh = plsc.VectorSubcoreMesh(core_axis_name="c", subcore_axis_name="s")
    kfn = pl.kernel(
        _sc_body, mesh=mesh,
        compiler_params=pltpu.CompilerParams(needs_layout_passes=False),
        out_type=[
            jax.ShapeDtypeStruct((L,), jnp.int32),
            jax.ShapeDtypeStruct((L,), jnp.float32),
            jax.ShapeDtypeStruct((RP_FLAT,), jnp.float32),
        ],
        scratch_types=[
            pltpu.VMEM((LCHUNK,), jnp.float32),
            pltpu.VMEM((RP_CHUNK,), jnp.float32),
            pltpu.VMEM((L,), jnp.float32),
            pltpu.VMEM((L,), jnp.float32),
            pltpu.VMEM((L,), jnp.int32),
            pltpu.VMEM((16 * L,), jnp.float32),
            pltpu.VMEM((16 * L,), jnp.int32),
            pltpu.VMEM_SHARED((16 * L,), jnp.float32),
            pltpu.VMEM_SHARED((16 * L,), jnp.int32),
        ],
    )
    out_i, out_v, rp_out = kfn(lflat, pen16, rp_pad)
    top_idx = out_i[:BEAM].reshape(BEAM, 1)
    top_prob = out_v[:BEAM].reshape(BEAM, 1)
    rp_final = rp_out.reshape(BEAM, RP_W)[:, :VOCAB]
    return top_idx, top_prob, rp_final


def _dma_body(*refs):
    kv_in = refs[:NKV]
    kv_out = refs[NKV:2 * NKV]
    vbufs = refs[2 * NKV:2 * NKV + NSLOTS]
    in_sems = refs[2 * NKV + NSLOTS]
    out_sems = refs[2 * NKV + NSLOTS + 1]

    def in_copy(i):
        s = i % NSLOTS
        return pltpu.make_async_copy(kv_in[i].at[0], vbufs[s], in_sems.at[s])

    def out_copy(i, b):
        s = i % NSLOTS
        return pltpu.make_async_copy(
            vbufs[s], kv_out[i].at[b], out_sems.at[s, b])

    def wait_outs(i):
        for b in range(BEAM):
            out_copy(i, b).wait()

    for i in range(3):
        in_copy(i).start()
    in_copy(0).wait()
    for b in range(BEAM):
        out_copy(0, b).start()
    outs_waited = set()
    for i in range(1, NKV):
        in_copy(i).wait()
        for b in range(BEAM):
            out_copy(i, b).start()
        k = i + 2
        if k < NKV:
            if k - NSLOTS >= 0:
                wait_outs(k - NSLOTS)
                outs_waited.add(k - NSLOTS)
            in_copy(k).start()
    for i in range(NKV):
        if i not in outs_waited:
            wait_outs(i)


def _beam_tile(kvs):
    return pl.pallas_call(
        _dma_body,
        in_specs=[pl.BlockSpec(memory_space=pl.ANY)] * NKV,
        out_specs=[pl.BlockSpec(memory_space=pl.ANY)] * NKV,
        out_shape=[jax.ShapeDtypeStruct((BEAM, 8, 2048, 64), jnp.float32)] * NKV,
        scratch_shapes=(
            [pltpu.VMEM((8, 2048, 64), jnp.float32)] * NSLOTS
            + [pltpu.SemaphoreType.DMA((NSLOTS,)),
               pltpu.SemaphoreType.DMA((NSLOTS, BEAM))]
        ),
    )(*kvs)


def kernel(kv_0, kv_1, kv_2, kv_3, kv_4, kv_5, kv_6, kv_7, kv_8, kv_9,
           kv_10, kv_11, kv_12, kv_13, kv_14, kv_15,
           logits, save_id, repeat_penality, penality_value, beam_size):
    kvs = [kv_0, kv_1, kv_2, kv_3, kv_4, kv_5, kv_6, kv_7,
           kv_8, kv_9, kv_10, kv_11, kv_12, kv_13, kv_14, kv_15]
    saved = _beam_tile(kvs)
    top_idx, top_prob, rp_out = _sc_topk(
        logits, repeat_penality, penality_value)
    beam = save_id.shape[0]
    save_id_out = jnp.concatenate([save_id, top_idx], axis=-1)
    batch_indices = jnp.arange(beam, dtype=jnp.int32) + (
        jnp.asarray(beam_size, dtype=jnp.int32) - beam)
    max_logits_idx = top_idx[0]
    return (*saved, top_idx, save_id_out, rp_out, top_prob,
            batch_indices, max_logits_idx)


# speedup vs baseline: 1.0045x; 1.0045x over previous
import functools

import jax
import jax.numpy as jnp
from jax import lax
from jax.experimental import pallas as pl
from jax.experimental.pallas import tpu as pltpu
from jax.experimental.pallas import tpu_sc as plsc

BEAM = 4
VOCAB = 100000
NKV = 16
NEG_BIG = -1e30
BIGI = jnp.int32(2 ** 30)
NSLOTS = 5

L = 16
LPAD = 102400
LCHUNK = LPAD // 16
LVREGS = LCHUNK // L

RP_W = 100096
RP_FLAT = BEAM * RP_W
RP_CHUNK = RP_FLAT // 32
RP_VREGS = RP_CHUNK // L

_LN2 = 0.6931471805599453


def _sc_body(logits_hbm, pen_hbm, rp_hbm,
             out_i_hbm, out_v_hbm, rp_out_hbm,
             lbuf, rbuf, pen_v, row_v, irow_v, m2, i2,
             stage_v, stage_i):
    cid = lax.axis_index("c")
    sid = lax.axis_index("s")
    lane = lax.broadcasted_iota(jnp.int32, (L,), 0)

    pltpu.sync_copy(pen_hbm, pen_v)

    base_l = sid * LCHUNK
    pltpu.sync_copy(logits_hbm.at[pl.ds(base_l, LCHUNK)], lbuf)

    def pass1(j, c):
        bv, bi = c
        v = lbuf[pl.ds(j * L, L)]
        gi = base_l + j * L + lane
        upd = v > bv
        return jnp.where(upd, v, bv), jnp.where(upd, gi, bi)

    bv, bi = lax.fori_loop(
        0, LVREGS, pass1,
        (jnp.full((L,), NEG_BIG, jnp.float32), jnp.zeros((L,), jnp.int32)))
    m1 = jnp.max(bv)
    i1 = jnp.min(jnp.where(bv == m1, bi, BIGI))
    m_w = m1

    def pass_se(j, s):
        v = lbuf[pl.ds(j * L, L)]
        return s + jnp.exp(v - m_w)

    s_vec = lax.fori_loop(0, LVREGS, pass_se, jnp.zeros((L,), jnp.float32))
    s_w = jnp.sum(s_vec)

    found_v = [m1]
    found_i = [i1]
    for _ in range(1, BEAM):
        prev = list(found_i)

        def passt(j, c, prev=prev):
            bv2, bi2 = c
            v = lbuf[pl.ds(j * L, L)]
            gi = base_l + j * L + lane
            excl = gi == prev[0]
            for p in prev[1:]:
                excl = excl | (gi == p)
            v = jnp.where(excl, NEG_BIG, v)
            upd = v > bv2
            return jnp.where(upd, v, bv2), jnp.where(upd, gi, bi2)

        bv2, bi2 = lax.fori_loop(
            0, LVREGS, passt,
            (jnp.full((L,), NEG_BIG, jnp.float32), jnp.zeros((L,), jnp.int32)))
        mt = jnp.max(bv2)
        it = jnp.min(jnp.where(bv2 == mt, bi2, BIGI))
        found_v.append(mt)
        found_i.append(it)

    sv = jnp.zeros((L,), jnp.float32)
    si = jnp.zeros((L,), jnp.int32)
    for t in range(BEAM):
        sv = jnp.where(lane == t, found_v[t], sv)
        si = jnp.where(lane == t, found_i[t], si)
    sv = jnp.where(lane == BEAM, m_w, sv)
    sv = jnp.where(lane == BEAM + 1, s_w, sv)
    row_v[...] = sv
    irow_v[...] = si
    pltpu.sync_copy(row_v, stage_v.at[pl.ds(sid * L, L)])
    pltpu.sync_copy(irow_v, stage_i.at[pl.ds(sid * L, L)])
    plsc.subcore_barrier()

    @pl.when(sid == 0)
    def _merge():
        pltpu.sync_copy(stage_v, m2)
        pltpu.sync_copy(stage_i, i2)

        def gmax(r, acc):
            rowr = m2[pl.ds(r * L, L)]
            m_r = jnp.max(jnp.where(lane == BEAM, rowr, NEG_BIG))
            return jnp.maximum(acc, m_r)

        gm = lax.fori_loop(0, 16, gmax, jnp.float32(NEG_BIG))

        def gsum(r, acc):
            rowr = m2[pl.ds(r * L, L)]
            m_r = jnp.max(jnp.where(lane == BEAM, rowr, NEG_BIG))
            s_r = jnp.max(jnp.where(lane == BEAM + 1, rowr, NEG_BIG))
            e = jnp.exp((m_r - gm) + jnp.zeros((L,), jnp.float32))
            return acc + s_r * jnp.max(e)

        gs = lax.fori_loop(0, 16, gsum, jnp.float32(0.0))

        g_v = []
        g_i = []
        for _ in range(BEAM):
            prev = list(g_i)

            def passg(r, c, prev=prev):
                bv3, bi3 = c
                vrow = m2[pl.ds(r * L, L)]
                irow = i2[pl.ds(r * L, L)]
                cand = jnp.where(lane < BEAM, vrow, NEG_BIG)
                for p in prev:
                    cand = jnp.where(irow == p, NEG_BIG, cand)
                upd = cand > bv3
                return jnp.where(upd, cand, bv3), jnp.where(upd, irow, bi3)

            bv3, bi3 = lax.fori_loop(
                0, 16, passg,
                (jnp.full((L,), NEG_BIG, jnp.float32),
                 jnp.zeros((L,), jnp.int32)))
            mt = jnp.max(bv3)
            it = jnp.min(jnp.where(bv3 == mt, bi3, BIGI))
            g_v.append(mt)
            g_i.append(it)

        s_splat = gs + jnp.zeros((L,), jnp.float32)
        bits = lax.bitcast_convert_type(s_splat, jnp.int32)
        e0 = ((bits >> 23) & 255) - 127
        y = e0.astype(jnp.float32) * jnp.float32(_LN2)
        for _ in range(4):
            y = y - 1.0 + s_splat * jnp.exp(-y)
        lse = gm + jnp.max(y)

        ov = jnp.zeros((L,), jnp.float32)
        oi = jnp.zeros((L,), jnp.int32)
        for t in range(BEAM):
            ov = jnp.where(lane == t, g_v[t] - lse, ov)
            oi = jnp.where(lane == t, g_i[t], oi)
        row_v[...] = ov
        irow_v[...] = oi
        pltpu.sync_copy(irow_v, stage_i.at[pl.ds(0, L)])

        @pl.when(cid == 0)
        def _write_small():
            pltpu.sync_copy(row_v, out_v_hbm)
            pltpu.sync_copy(irow_v, out_i_hbm)

    plsc.subcore_barrier()

    pltpu.sync_copy(stage_i.at[pl.ds(0, L)], irow_v)
    irow = irow_v[...]
    j0 = jnp.max(jnp.where(lane == 0, irow, -1))
    j1 = jnp.max(jnp.where(lane == 1, irow, -1))
    j2 = jnp.max(jnp.where(lane == 2, irow, -1))
    j3 = jnp.max(jnp.where(lane == 3, irow, -1))
    pen = pen_v[...]

    chunk_id = sid * 2 + cid
    base_r = chunk_id * RP_CHUNK
    pltpu.sync_copy(rp_hbm.at[pl.ds(base_r, RP_CHUNK)], rbuf)

    def rp_pass(j, carry):
        v = rbuf[pl.ds(j * L, L)]
        g = base_r + j * L + lane
        colx = lax.rem(g, RP_W)
        hit = (colx == j0) | (colx == j1) | (colx == j2) | (colx == j3)
        rbuf[pl.ds(j * L, L)] = jnp.where(hit, v * pen, v)
        return carry

    lax.fori_loop(0, RP_VREGS, rp_pass, 0)
    pltpu.sync_copy(rbuf, rp_out_hbm.at[pl.ds(base_r, RP_CHUNK)])


def _sc_topk(logits, repeat_penality, penality_value):
    lflat = jnp.concatenate(
        [logits.reshape(VOCAB),
         jnp.full((LPAD - VOCAB,), NEG_BIG, jnp.float32)])
    pen16 = jnp.tile(penality_value, L)
    rp_pad = jnp.pad(repeat_penality, ((0, 0), (0, RP_W - VOCAB)),
                     constant_values=1.0).reshape(RP_FLAT)

    mesh = plsc.VectorSubcoreMesh(core_axis_name="c", subcore_axis_name="s")
    kfn = pl.kernel(
        _sc_body, mesh=mesh,
        compiler_params=pltpu.CompilerParams(needs_layout_passes=False),
        out_type=[
            jax.ShapeDtypeStruct((L,), jnp.int32),
            jax.ShapeDtypeStruct((L,), jnp.float32),
            jax.ShapeDtypeStruct((RP_FLAT,), jnp.float32),
        ],
        scratch_types=[
            pltpu.VMEM((LCHUNK,), jnp.float32),
            pltpu.VMEM((RP_CHUNK,), jnp.float32),
            pltpu.VMEM((L,), jnp.float32),
            pltpu.VMEM((L,), jnp.float32),
            pltpu.VMEM((L,), jnp.int32),
            pltpu.VMEM((16 * L,), jnp.float32),
            pltpu.VMEM((16 * L,), jnp.int32),
            pltpu.VMEM_SHARED((16 * L,), jnp.float32),
            pltpu.VMEM_SHARED((16 * L,), jnp.int32),
        ],
    )
    out_i, out_v, rp_out = kfn(lflat, pen16, rp_pad)
    top_idx = out_i[:BEAM].reshape(BEAM, 1)
    top_prob = out_v[:BEAM].reshape(BEAM, 1)
    rp_final = rp_out.reshape(BEAM, RP_W)[:, :VOCAB]
    return top_idx, top_prob, rp_final


def _dma_body(*refs):
    kv_in = refs[:NKV]
    kv_out = refs[NKV:2 * NKV]
    vbufs = refs[2 * NKV:2 * NKV + NSLOTS]
    in_sems = refs[2 * NKV + NSLOTS]
    out_sems = refs[2 * NKV + NSLOTS + 1]

    def in_copy(i):
        s = i % NSLOTS
        return pltpu.make_async_copy(kv_in[i].at[0], vbufs[s], in_sems.at[s])

    def out_copy(i, b):
        s = i % NSLOTS
        return pltpu.make_async_copy(
            vbufs[s], kv_out[i].at[b], out_sems.at[s, b])

    def wait_outs(i):
        for b in range(BEAM):
            out_copy(i, b).wait()

    for i in range(3):
        in_copy(i).start(priority=1)
    in_copy(0).wait()
    for b in range(BEAM):
        out_copy(0, b).start()
    outs_waited = set()
    for i in range(1, NKV):
        in_copy(i).wait()
        for b in range(BEAM):
            out_copy(i, b).start()
        k = i + 2
        if k < NKV:
            if k - NSLOTS >= 0:
                wait_outs(k - NSLOTS)
                outs_waited.add(k - NSLOTS)
            in_copy(k).start(priority=1)
    for i in range(NKV):
        if i not in outs_waited:
            wait_outs(i)


def _beam_tile(kvs):
    return pl.pallas_call(
        _dma_body,
        in_specs=[pl.BlockSpec(memory_space=pl.ANY)] * NKV,
        out_specs=[pl.BlockSpec(memory_space=pl.ANY)] * NKV,
        out_shape=[jax.ShapeDtypeStruct((BEAM, 8, 2048, 64), jnp.float32)] * NKV,
        scratch_shapes=(
            [pltpu.VMEM((8, 2048, 64), jnp.float32)] * NSLOTS
            + [pltpu.SemaphoreType.DMA((NSLOTS,)),
               pltpu.SemaphoreType.DMA((NSLOTS, BEAM))]
        ),
    )(*kvs)


def kernel(kv_0, kv_1, kv_2, kv_3, kv_4, kv_5, kv_6, kv_7, kv_8, kv_9,
           kv_10, kv_11, kv_12, kv_13, kv_14, kv_15,
           logits, save_id, repeat_penality, penality_value, beam_size):
    kvs = [kv_0, kv_1, kv_2, kv_3, kv_4, kv_5, kv_6, kv_7,
           kv_8, kv_9, kv_10, kv_11, kv_12, kv_13, kv_14, kv_15]
    saved = _beam_tile(kvs)
    top_idx, top_prob, rp_out = _sc_topk(
        logits, repeat_penality, penality_value)
    beam = save_id.shape[0]
    save_id_out = jnp.concatenate([save_id, top_idx], axis=-1)
    batch_indices = jnp.arange(beam, dtype=jnp.int32) + (
        jnp.asarray(beam_size, dtype=jnp.int32) - beam)
    max_logits_idx = top_idx[0]
    return (*saved, top_idx, save_id_out, rp_out, top_prob,
            batch_indices, max_logits_idx)
